# baseline (speedup 1.0000x reference)
import functools

import jax
import jax.numpy as jnp
import numpy as np
from jax import lax
from jax.experimental import pallas as pl
from jax.experimental.pallas import tpu as pltpu
from jax.experimental.pallas import tpu_sc as plsc

N = 10000
E = 160000
H = 16
NSTEPS = 5

_CHUNK = 128
_ROWS = E // _CHUNK
_NW = 32
_RPW = _ROWS // _NW
_XTRA = _ROWS - _RPW * _NW
_EW = _RPW * _CHUNK
_NPS = N // 16

_NP = N // 8
_EP = E // 8
_BE = 2000

_SK = np.zeros((128, H * 128), np.float32)
for _a in range(8):
    for _k in range(H):
        _SK[_a * H + _k, _k * 128 + _a * H:_k * 128 + (_a + 1) * H] = 1.0
_FOLD = np.tile(np.eye(H, dtype=np.float32), (8, 1))
_EYE8 = np.eye(8, dtype=np.float32)


def _dot(a, b):
    return jax.lax.dot_general(a, b, (((1,), (0,)), ((), ())),
                               preferred_element_type=jnp.float32)


def _tile8(b, g):
    return jnp.broadcast_to(b.reshape(g, 1, H), (g, 8, H)).reshape(1, g * 128)


def _affine_relu_body(x_ref, w_ref, b_ref, o_ref):
    o_ref[...] = jax.nn.relu(_dot(x_ref[...], w_ref[...]) + b_ref[...])


def _affine_relu_packed(x, w, b, bm):
    m, k = x.shape
    n = w.shape[1]
    return pl.pallas_call(
        _affine_relu_body,
        grid=(m // bm,),
        in_specs=[pl.BlockSpec((bm, k), lambda i: (i, 0)),
                  pl.BlockSpec((k, n), lambda i: (0, 0)),
                  pl.BlockSpec((1, n), lambda i: (0, 0))],
        out_specs=pl.BlockSpec((bm, n), lambda i: (i, 0)),
        out_shape=jax.ShapeDtypeStruct((m, n), jnp.float32),
    )(x, w, b)


def _msg_body(u_ref, hs_ref, sk_ref, gk_ref, bdbe2_ref, o_ref):
    u = u_ref[...]
    x = hs_ref[...]
    us = _dot(u, sk_ref[...])
    xg = _dot(x, gk_ref[...])
    acc = _dot(x, bdbe2_ref[...])
    for k in range(H):
        acc = acc + us[:, k * 128:(k + 1) * 128] * xg[:, k * 128:(k + 1) * 128]
    o_ref[...] = acc


def _msg(u_p, hs_p, sk, gk, bdbe2):
    return pl.pallas_call(
        _msg_body,
        grid=(_EP // _BE,),
        in_specs=[pl.BlockSpec((_BE, 128), lambda i: (i, 0)),
                  pl.BlockSpec((_BE, 128), lambda i: (i, 0)),
                  pl.BlockSpec((128, H * 128), lambda i: (0, 0)),
                  pl.BlockSpec((128, H * 128), lambda i: (0, 0)),
                  pl.BlockSpec((128, 128), lambda i: (0, 0))],
        out_specs=pl.BlockSpec((_BE, 128), lambda i: (i, 0)),
        out_shape=jax.ShapeDtypeStruct((_EP, 128), jnp.float32),
    )(u_p, hs_p, sk, gk, bdbe2)


def _gru_body(parts_ref, hid_ref, wi_ref, wh_ref, bi_ref, bh_ref,
              bnn_ref, o_ref):
    m = jax.nn.relu(parts_ref[0] + parts_ref[1] + bnn_ref[...])
    hid = hid_ref[...]
    gi = _dot(m, wi_ref[...]) + bi_ref[...]
    gh = _dot(hid, wh_ref[...]) + bh_ref[...]
    r = jax.nn.sigmoid(gi[:, 0:128] + gh[:, 0:128])
    z = jax.nn.sigmoid(gi[:, 128:256] + gh[:, 128:256])
    ng = jnp.tanh(gi[:, 256:384] + r * gh[:, 256:384])
    o_ref[...] = (1.0 - z) * ng + z * hid


def _gru(parts_p3, hid_p, bdi, bdh, bi3, bh3, bnn128):
    return pl.pallas_call(
        _gru_body,
        grid=(1,),
        in_specs=[pl.BlockSpec((2, _NP, 128), lambda i: (0, 0, 0)),
                  pl.BlockSpec((_NP, 128), lambda i: (0, 0)),
                  pl.BlockSpec((128, 384), lambda i: (0, 0)),
                  pl.BlockSpec((128, 384), lambda i: (0, 0)),
                  pl.BlockSpec((1, 384), lambda i: (0, 0)),
                  pl.BlockSpec((1, 384), lambda i: (0, 0)),
                  pl.BlockSpec((1, 128), lambda i: (0, 0))],
        out_specs=pl.BlockSpec((_NP, 128), lambda i: (0, 0)),
        out_shape=jax.ShapeDtypeStruct((_NP, 128), jnp.float32),
    )(parts_p3, hid_p, bdi, bdh, bi3, bh3, bnn128)


def _readout_body(h_ref, fold_ref, w0_ref, b0_ref, w1_ref, b1_ref, w2_ref,
                  b2_ref, y_ref):
    s = jnp.sum(h_ref[...], axis=0, keepdims=True)
    hg = _dot(s, fold_ref[...]) * (1.0 / N)
    y = jax.nn.relu(_dot(hg, w0_ref[...]) + b0_ref[...])
    y = jax.nn.relu(_dot(y, w1_ref[...]) + b1_ref[...])
    y_ref[...] = _dot(y, w2_ref[...]) + b2_ref[...]


def _readout(h_p, fold, w0, b0, w1, b1, w2, b2):
    return pl.pallas_call(
        _readout_body,
        grid=(1,),
        in_specs=[pl.BlockSpec((_NP, 128), lambda i: (0, 0)),
                  pl.BlockSpec((128, H), lambda i: (0, 0)),
                  pl.BlockSpec(w0.shape, lambda i: (0, 0)),
                  pl.BlockSpec((1, w0.shape[1]), lambda i: (0, 0)),
                  pl.BlockSpec(w1.shape, lambda i: (0, 0)),
                  pl.BlockSpec((1, w1.shape[1]), lambda i: (0, 0)),
                  pl.BlockSpec(w2.shape, lambda i: (0, 0)),
                  pl.BlockSpec((1, w2.shape[1]), lambda i: (0, 0))],
        out_specs=pl.BlockSpec((1, 2), lambda i: (0, 0)),
        out_shape=jax.ShapeDtypeStruct((1, 2), jnp.float32),
    )(h_p, fold, w0, b0.reshape(1, -1), w1, b1.reshape(1, -1), w2,
      b2.reshape(1, -1))


_MESH = plsc.VectorSubcoreMesh(core_axis_name="c", subcore_axis_name="s")
_SC_PARAMS = pltpu.CompilerParams(use_tc_tiling_on_sc=False)


@functools.partial(
    pl.kernel,
    out_type=jax.ShapeDtypeStruct((E, H), jnp.float32),
    mesh=_MESH,
    compiler_params=_SC_PARAMS,
    scratch_types=[pltpu.VMEM((_RPW, _CHUNK), jnp.int32),
                   pltpu.VMEM((_EW, H), jnp.float32),
                   pltpu.VMEM((1, _CHUNK), jnp.int32),
                   pltpu.VMEM((_CHUNK, H), jnp.float32),
                   pltpu.SemaphoreType.DMA,
                   pltpu.SemaphoreType.DMA],
)
def _sc_gather(h_hbm, src_hbm, out_hbm, idx_v, rows_v, idx_x, rows_x,
               sem, sem_x):
    wid = lax.axis_index("s") * 2 + lax.axis_index("c")
    base = wid * _RPW

    pltpu.sync_copy(src_hbm.at[pl.ds(base, _RPW)], idx_v)

    def fire(j, carry):
        pltpu.async_copy(h_hbm.at[idx_v.at[j]],
                         rows_v.at[pl.ds(j * _CHUNK, _CHUNK)], sem)
        return carry

    lax.fori_loop(0, _RPW, fire, 0)

    @pl.when(wid < _XTRA)
    def _():
        r = _RPW * _NW + wid
        pltpu.sync_copy(src_hbm.at[pl.ds(r, 1)], idx_x)
        pltpu.async_copy(h_hbm.at[idx_x.at[0]], rows_x, sem_x)

    pltpu.make_async_copy(out_hbm.at[pl.ds(0, _EW)], rows_v, sem).wait()
    pltpu.sync_copy(rows_v, out_hbm.at[pl.ds(base * _CHUNK, _EW)])

    @pl.when(wid < _XTRA)
    def _():
        r = _RPW * _NW + wid
        pltpu.make_async_copy(out_hbm.at[pl.ds(0, _CHUNK)], rows_x,
                              sem_x).wait()
        pltpu.sync_copy(rows_x, out_hbm.at[pl.ds(r * _CHUNK, _CHUNK)])


@functools.partial(
    pl.kernel,
    out_type=jax.ShapeDtypeStruct((2 * N, H), jnp.float32),
    mesh=_MESH,
    compiler_params=_SC_PARAMS,
    scratch_types=[pltpu.VMEM((_RPW, _CHUNK), jnp.int32),
                   pltpu.VMEM((_EW, H), jnp.float32),
                   pltpu.VMEM((1, _CHUNK), jnp.int32),
                   pltpu.VMEM((_CHUNK, H), jnp.float32),
                   pltpu.VMEM_SHARED((N, H), jnp.float32),
                   pltpu.SemaphoreType.DMA,
                   pltpu.SemaphoreType.DMA],
)
def _sc_scatter(msg_hbm, dst_hbm, zeros_hbm, out_hbm, idx_v, msg_v, idx_x,
                msg_x, acc_sh, sem, sem_x):
    c = lax.axis_index("c")
    s = lax.axis_index("s")
    wid = s * 2 + c
    base = wid * _RPW

    pltpu.async_copy(zeros_hbm.at[pl.ds(s * _NPS, _NPS)],
                     acc_sh.at[pl.ds(s * _NPS, _NPS)], sem_x)
    pltpu.sync_copy(dst_hbm.at[pl.ds(base, _RPW)], idx_v)
    pltpu.sync_copy(msg_hbm.at[pl.ds(base * _CHUNK, _EW)], msg_v)
    pltpu.make_async_copy(zeros_hbm.at[pl.ds(0, _NPS)],
                          acc_sh.at[pl.ds(0, _NPS)], sem_x).wait()
    plsc.subcore_barrier()

    def fire(j, carry):
        pltpu.async_copy(msg_v.at[pl.ds(j * _CHUNK, _CHUNK)],
                         acc_sh.at[idx_v.at[j]], sem, add=True)
        return carry

    lax.fori_loop(0, _RPW, fire, 0)

    @pl.when(wid < _XTRA)
    def _():
        r = _RPW * _NW + wid
        pltpu.sync_copy(dst_hbm.at[pl.ds(r, 1)], idx_x)
        pltpu.sync_copy(msg_hbm.at[pl.ds(r * _CHUNK, _CHUNK)], msg_x)
        pltpu.async_copy(msg_x, acc_sh.at[idx_x.at[0]], sem, add=True)

    pltpu.make_async_copy(msg_hbm.at[pl.ds(0, _EW)], msg_v, sem).wait()

    @pl.when(wid < _XTRA)
    def _():
        pltpu.make_async_copy(msg_hbm.at[pl.ds(0, _CHUNK)], msg_x,
                              sem).wait()

    plsc.subcore_barrier()
    pltpu.sync_copy(acc_sh.at[pl.ds(s * _NPS, _NPS)],
                    out_hbm.at[pl.ds(c * N + s * _NPS, _NPS)])


def kernel(node_feats, edge_feats, edge_index, W_proj, b_proj, We1, be1,
           We2, be2, b_nn, W_ih, W_hh, b_ih, b_hh, W0, b0, W1, b1, W2, b2):
    f32 = jnp.float32
    src2d = edge_index[0].reshape(_ROWS, _CHUNK)
    dst2d = edge_index[1].reshape(_ROWS, _CHUNK)
    zeros = jnp.zeros((N, H), f32)
    eye8 = jnp.asarray(_EYE8)
    sk = jnp.asarray(_SK)
    fold = jnp.asarray(_FOLD)

    we2r = We2.reshape(H, H, H)
    gk = jnp.einsum('ab,kio->aikbo', eye8, we2r).reshape(128, H * 128)
    bdbe2 = jnp.einsum('ab,io->aibo', eye8,
                       be2.reshape(H, H)).reshape(128, 128)

    wi3 = W_ih.reshape(3, H, H)
    wh3 = W_hh.reshape(3, H, H)
    bdi = jnp.einsum('ab,goi->aigbo', eye8, wi3).reshape(128, 384)
    bdh = jnp.einsum('ab,goi->aigbo', eye8, wh3).reshape(128, 384)
    bi3 = _tile8(b_ih, 3)
    bh3 = _tile8(b_hh, 3)
    bnn128 = _tile8(b_nn, 1)

    bdn = jnp.einsum('ab,ko->akbo', eye8, W_proj).reshape(1024, 128)
    bde = jnp.einsum('ab,ko->akbo', eye8, We1).reshape(128, 128)
    bproj128 = _tile8(b_proj, 1)
    be1_128 = _tile8(be1, 1)

    nf8 = node_feats.reshape(_NP, 8 * 128)
    ef8 = edge_feats.reshape(_EP, 128)

    h_p = _affine_relu_packed(nf8, bdn, bproj128, _NP)
    u_p = _affine_relu_packed(ef8, bde, be1_128, 4000)
    hid_p = h_p
    for _ in range(NSTEPS):
        h_rows = h_p.reshape(N, H)
        h_src = _sc_gather(h_rows, src2d)
        msg_p = _msg(u_p, h_src.reshape(_EP, 128), sk, gk, bdbe2)
        parts = _sc_scatter(msg_p.reshape(E, H), dst2d, zeros)
        hid_p = _gru(parts.reshape(2, _NP, 128), hid_p, bdi, bdh, bi3,
                     bh3, bnn128)
        h_p = hid_p
    return _readout(h_p, fold, W0, b0, W1, b1, W2, b2)


# speedup vs baseline: 1.0076x; 1.0076x over previous
import functools

import jax
import jax.numpy as jnp
import numpy as np
from jax import lax
from jax.experimental import pallas as pl
from jax.experimental.pallas import tpu as pltpu
from jax.experimental.pallas import tpu_sc as plsc

N = 10000
E = 160000
H = 16
NSTEPS = 5

_CHUNK = 128
_ROWS = E // _CHUNK
_NW = 32
_RPW = _ROWS // _NW
_XTRA = _ROWS - _RPW * _NW
_EW = _RPW * _CHUNK
_NPS = N // 16

_NP = N // 8
_EP = E // 8
_BE = 1000

_SK = np.zeros((128, H * 128), np.float32)
for _a in range(8):
    for _k in range(H):
        _SK[_a * H + _k, _k * 128 + _a * H:_k * 128 + (_a + 1) * H] = 1.0
_FOLD = np.tile(np.eye(H, dtype=np.float32), (8, 1))
_EYE8 = np.eye(8, dtype=np.float32)


def _dot(a, b):
    return jax.lax.dot_general(a, b, (((1,), (0,)), ((), ())),
                               preferred_element_type=jnp.float32)


def _tile8(b, g):
    return jnp.broadcast_to(b.reshape(g, 1, H), (g, 8, H)).reshape(1, g * 128)


def _affine_relu_body(x_ref, w_ref, b_ref, o_ref):
    o_ref[...] = jax.nn.relu(_dot(x_ref[...], w_ref[...]) + b_ref[...])


def _affine_relu_packed(x, w, b, bm):
    m, k = x.shape
    n = w.shape[1]
    return pl.pallas_call(
        _affine_relu_body,
        grid=(m // bm,),
        in_specs=[pl.BlockSpec((bm, k), lambda i: (i, 0)),
                  pl.BlockSpec((k, n), lambda i: (0, 0)),
                  pl.BlockSpec((1, n), lambda i: (0, 0))],
        out_specs=pl.BlockSpec((bm, n), lambda i: (i, 0)),
        out_shape=jax.ShapeDtypeStruct((m, n), jnp.float32),
    )(x, w, b)


def _msg_body(u_ref, hs_ref, sk_ref, gk_ref, bdbe2_ref, o_ref):
    u = u_ref[...]
    x = hs_ref[...]
    us = _dot(u, sk_ref[...])
    xg = _dot(x, gk_ref[...])
    acc = _dot(x, bdbe2_ref[...])
    for k in range(H):
        acc = acc + us[:, k * 128:(k + 1) * 128] * xg[:, k * 128:(k + 1) * 128]
    o_ref[...] = acc


def _msg(u_p, hs_p, sk, gk, bdbe2):
    return pl.pallas_call(
        _msg_body,
        grid=(_EP // _BE,),
        in_specs=[pl.BlockSpec((_BE, 128), lambda i: (i, 0)),
                  pl.BlockSpec((_BE, 128), lambda i: (i, 0)),
                  pl.BlockSpec((128, H * 128), lambda i: (0, 0)),
                  pl.BlockSpec((128, H * 128), lambda i: (0, 0)),
                  pl.BlockSpec((128, 128), lambda i: (0, 0))],
        out_specs=pl.BlockSpec((_BE, 128), lambda i: (i, 0)),
        out_shape=jax.ShapeDtypeStruct((_EP, 128), jnp.float32),
    )(u_p, hs_p, sk, gk, bdbe2)


def _gru_body(parts_ref, hid_ref, wi_ref, wh_ref, bi_ref, bh_ref,
              bnn_ref, o_ref):
    m = jax.nn.relu(parts_ref[0] + parts_ref[1] + bnn_ref[...])
    hid = hid_ref[...]
    gi = _dot(m, wi_ref[...]) + bi_ref[...]
    gh = _dot(hid, wh_ref[...]) + bh_ref[...]
    r = jax.nn.sigmoid(gi[:, 0:128] + gh[:, 0:128])
    z = jax.nn.sigmoid(gi[:, 128:256] + gh[:, 128:256])
    ng = jnp.tanh(gi[:, 256:384] + r * gh[:, 256:384])
    o_ref[...] = (1.0 - z) * ng + z * hid


def _gru(parts_p3, hid_p, bdi, bdh, bi3, bh3, bnn128):
    return pl.pallas_call(
        _gru_body,
        grid=(1,),
        in_specs=[pl.BlockSpec((2, _NP, 128), lambda i: (0, 0, 0)),
                  pl.BlockSpec((_NP, 128), lambda i: (0, 0)),
                  pl.BlockSpec((128, 384), lambda i: (0, 0)),
                  pl.BlockSpec((128, 384), lambda i: (0, 0)),
                  pl.BlockSpec((1, 384), lambda i: (0, 0)),
                  pl.BlockSpec((1, 384), lambda i: (0, 0)),
                  pl.BlockSpec((1, 128), lambda i: (0, 0))],
        out_specs=pl.BlockSpec((_NP, 128), lambda i: (0, 0)),
        out_shape=jax.ShapeDtypeStruct((_NP, 128), jnp.float32),
    )(parts_p3, hid_p, bdi, bdh, bi3, bh3, bnn128)


def _readout_body(h_ref, fold_ref, w0_ref, b0_ref, w1_ref, b1_ref, w2_ref,
                  b2_ref, y_ref):
    s = jnp.sum(h_ref[...], axis=0, keepdims=True)
    hg = _dot(s, fold_ref[...]) * (1.0 / N)
    y = jax.nn.relu(_dot(hg, w0_ref[...]) + b0_ref[...])
    y = jax.nn.relu(_dot(y, w1_ref[...]) + b1_ref[...])
    y_ref[...] = _dot(y, w2_ref[...]) + b2_ref[...]


def _readout(h_p, fold, w0, b0, w1, b1, w2, b2):
    return pl.pallas_call(
        _readout_body,
        grid=(1,),
        in_specs=[pl.BlockSpec((_NP, 128), lambda i: (0, 0)),
                  pl.BlockSpec((128, H), lambda i: (0, 0)),
                  pl.BlockSpec(w0.shape, lambda i: (0, 0)),
                  pl.BlockSpec((1, w0.shape[1]), lambda i: (0, 0)),
                  pl.BlockSpec(w1.shape, lambda i: (0, 0)),
                  pl.BlockSpec((1, w1.shape[1]), lambda i: (0, 0)),
                  pl.BlockSpec(w2.shape, lambda i: (0, 0)),
                  pl.BlockSpec((1, w2.shape[1]), lambda i: (0, 0))],
        out_specs=pl.BlockSpec((1, 2), lambda i: (0, 0)),
        out_shape=jax.ShapeDtypeStruct((1, 2), jnp.float32),
    )(h_p, fold, w0, b0.reshape(1, -1), w1, b1.reshape(1, -1), w2,
      b2.reshape(1, -1))


_MESH = plsc.VectorSubcoreMesh(core_axis_name="c", subcore_axis_name="s")
_SC_PARAMS = pltpu.CompilerParams(use_tc_tiling_on_sc=False)


@functools.partial(
    pl.kernel,
    out_type=jax.ShapeDtypeStruct((E, H), jnp.float32),
    mesh=_MESH,
    compiler_params=_SC_PARAMS,
    scratch_types=[pltpu.VMEM((_RPW, _CHUNK), jnp.int32),
                   pltpu.VMEM((_EW, H), jnp.float32),
                   pltpu.VMEM((1, _CHUNK), jnp.int32),
                   pltpu.VMEM((_CHUNK, H), jnp.float32),
                   pltpu.SemaphoreType.DMA,
                   pltpu.SemaphoreType.DMA,
                   pltpu.SemaphoreType.DMA,
                   pltpu.SemaphoreType.DMA],
)
def _sc_gather(h_hbm, src_hbm, out_hbm, idx_v, rows_v, idx_x, rows_x,
               sem_a, sem_b, sem_o, sem_x):
    wid = lax.axis_index("s") * 2 + lax.axis_index("c")
    base = wid * _RPW
    half = _RPW // 2
    e1 = half * _CHUNK
    e2 = _EW - e1

    pltpu.sync_copy(src_hbm.at[pl.ds(base, _RPW)], idx_v)

    def fire_a(j, carry):
        pltpu.async_copy(h_hbm.at[idx_v.at[j]],
                         rows_v.at[pl.ds(j * _CHUNK, _CHUNK)], sem_a)
        return carry

    def fire_b(j, carry):
        pltpu.async_copy(h_hbm.at[idx_v.at[j]],
                         rows_v.at[pl.ds(j * _CHUNK, _CHUNK)], sem_b)
        return carry

    lax.fori_loop(0, half, fire_a, 0)
    lax.fori_loop(half, _RPW, fire_b, 0)

    @pl.when(wid < _XTRA)
    def _():
        r = _RPW * _NW + wid
        pltpu.sync_copy(src_hbm.at[pl.ds(r, 1)], idx_x)
        pltpu.async_copy(h_hbm.at[idx_x.at[0]], rows_x, sem_x)

    pltpu.make_async_copy(out_hbm.at[pl.ds(0, e1)],
                          rows_v.at[pl.ds(0, e1)], sem_a).wait()
    pltpu.async_copy(rows_v.at[pl.ds(0, e1)],
                     out_hbm.at[pl.ds(base * _CHUNK, e1)], sem_o)
    pltpu.make_async_copy(out_hbm.at[pl.ds(0, e2)],
                          rows_v.at[pl.ds(e1, e2)], sem_b).wait()
    pltpu.sync_copy(rows_v.at[pl.ds(e1, e2)],
                    out_hbm.at[pl.ds(base * _CHUNK + e1, e2)])
    pltpu.make_async_copy(rows_v.at[pl.ds(0, e1)],
                          out_hbm.at[pl.ds(base * _CHUNK, e1)], sem_o).wait()

    @pl.when(wid < _XTRA)
    def _():
        r = _RPW * _NW + wid
        pltpu.make_async_copy(out_hbm.at[pl.ds(0, _CHUNK)], rows_x,
                              sem_x).wait()
        pltpu.sync_copy(rows_x, out_hbm.at[pl.ds(r * _CHUNK, _CHUNK)])


@functools.partial(
    pl.kernel,
    out_type=jax.ShapeDtypeStruct((2 * N, H), jnp.float32),
    mesh=_MESH,
    compiler_params=_SC_PARAMS,
    scratch_types=[pltpu.VMEM((_RPW, _CHUNK), jnp.int32),
                   pltpu.VMEM((_EW, H), jnp.float32),
                   pltpu.VMEM((1, _CHUNK), jnp.int32),
                   pltpu.VMEM((_CHUNK, H), jnp.float32),
                   pltpu.VMEM_SHARED((N, H), jnp.float32),
                   pltpu.SemaphoreType.DMA,
                   pltpu.SemaphoreType.DMA],
)
def _sc_scatter(msg_hbm, dst_hbm, zeros_hbm, out_hbm, idx_v, msg_v, idx_x,
                msg_x, acc_sh, sem, sem_x):
    c = lax.axis_index("c")
    s = lax.axis_index("s")
    wid = s * 2 + c
    base = wid * _RPW

    pltpu.async_copy(zeros_hbm.at[pl.ds(s * _NPS, _NPS)],
                     acc_sh.at[pl.ds(s * _NPS, _NPS)], sem_x)
    pltpu.sync_copy(dst_hbm.at[pl.ds(base, _RPW)], idx_v)
    pltpu.sync_copy(msg_hbm.at[pl.ds(base * _CHUNK, _EW)], msg_v)
    pltpu.make_async_copy(zeros_hbm.at[pl.ds(0, _NPS)],
                          acc_sh.at[pl.ds(0, _NPS)], sem_x).wait()
    plsc.subcore_barrier()

    def fire(j, carry):
        pltpu.async_copy(msg_v.at[pl.ds(j * _CHUNK, _CHUNK)],
                         acc_sh.at[idx_v.at[j]], sem, add=True)
        return carry

    lax.fori_loop(0, _RPW, fire, 0)

    @pl.when(wid < _XTRA)
    def _():
        r = _RPW * _NW + wid
        pltpu.sync_copy(dst_hbm.at[pl.ds(r, 1)], idx_x)
        pltpu.sync_copy(msg_hbm.at[pl.ds(r * _CHUNK, _CHUNK)], msg_x)
        pltpu.async_copy(msg_x, acc_sh.at[idx_x.at[0]], sem, add=True)

    pltpu.make_async_copy(msg_hbm.at[pl.ds(0, _EW)], msg_v, sem).wait()

    @pl.when(wid < _XTRA)
    def _():
        pltpu.make_async_copy(msg_hbm.at[pl.ds(0, _CHUNK)], msg_x,
                              sem).wait()

    plsc.subcore_barrier()
    pltpu.sync_copy(acc_sh.at[pl.ds(s * _NPS, _NPS)],
                    out_hbm.at[pl.ds(c * N + s * _NPS, _NPS)])


def kernel(node_feats, edge_feats, edge_index, W_proj, b_proj, We1, be1,
           We2, be2, b_nn, W_ih, W_hh, b_ih, b_hh, W0, b0, W1, b1, W2, b2):
    f32 = jnp.float32
    src2d = edge_index[0].reshape(_ROWS, _CHUNK)
    dst2d = edge_index[1].reshape(_ROWS, _CHUNK)
    zeros = jnp.zeros((N, H), f32)
    eye8 = jnp.asarray(_EYE8)
    sk = jnp.asarray(_SK)
    fold = jnp.asarray(_FOLD)

    we2r = We2.reshape(H, H, H)
    gk = jnp.einsum('ab,kio->aikbo', eye8, we2r).reshape(128, H * 128)
    bdbe2 = jnp.einsum('ab,io->aibo', eye8,
                       be2.reshape(H, H)).reshape(128, 128)

    wi3 = W_ih.reshape(3, H, H)
    wh3 = W_hh.reshape(3, H, H)
    bdi = jnp.einsum('ab,goi->aigbo', eye8, wi3).reshape(128, 384)
    bdh = jnp.einsum('ab,goi->aigbo', eye8, wh3).reshape(128, 384)
    bi3 = _tile8(b_ih, 3)
    bh3 = _tile8(b_hh, 3)
    bnn128 = _tile8(b_nn, 1)

    bdn = jnp.einsum('ab,ko->akbo', eye8, W_proj).reshape(1024, 128)
    bde = jnp.einsum('ab,ko->akbo', eye8, We1).reshape(128, 128)
    bproj128 = _tile8(b_proj, 1)
    be1_128 = _tile8(be1, 1)

    nf8 = node_feats.reshape(_NP, 8 * 128)
    ef8 = edge_feats.reshape(_EP, 128)

    h_p = _affine_relu_packed(nf8, bdn, bproj128, _NP)
    u_p = _affine_relu_packed(ef8, bde, be1_128, 4000)
    hid_p = h_p
    for _ in range(NSTEPS):
        h_rows = h_p.reshape(N, H)
        h_src = _sc_gather(h_rows, src2d)
        msg_p = _msg(u_p, h_src.reshape(_EP, 128), sk, gk, bdbe2)
        parts = _sc_scatter(msg_p.reshape(E, H), dst2d, zeros)
        hid_p = _gru(parts.reshape(2, _NP, 128), hid_p, bdi, bdh, bi3,
                     bh3, bnn128)
        h_p = hid_p
    return _readout(h_p, fold, W0, b0, W1, b1, W2, b2)


# speedup vs baseline: 1.0101x; 1.0026x over previous
import functools

import jax
import jax.numpy as jnp
import numpy as np
from jax import lax
from jax.experimental import pallas as pl
from jax.experimental.pallas import tpu as pltpu
from jax.experimental.pallas import tpu_sc as plsc

N = 10000
E = 160000
H = 16
NSTEPS = 5

_CHUNK = 128
_ROWS = E // _CHUNK
_NW = 32
_RPW = _ROWS // _NW
_XTRA = _ROWS - _RPW * _NW
_EW = _RPW * _CHUNK
_NPS = N // 16

_NP = N // 8
_EP = E // 8
_BE = 1000

_SK = np.zeros((128, H * 128), np.float32)
for _a in range(8):
    for _k in range(H):
        _SK[_a * H + _k, _k * 128 + _a * H:_k * 128 + (_a + 1) * H] = 1.0
_FOLD = np.tile(np.eye(H, dtype=np.float32), (8, 1))
_EYE8 = np.eye(8, dtype=np.float32)


def _dot(a, b):
    return jax.lax.dot_general(a, b, (((1,), (0,)), ((), ())),
                               preferred_element_type=jnp.float32)


def _tile8(b, g):
    return jnp.broadcast_to(b.reshape(g, 1, H), (g, 8, H)).reshape(1, g * 128)


def _affine_relu_body(x_ref, w_ref, b_ref, o_ref):
    o_ref[...] = jax.nn.relu(_dot(x_ref[...], w_ref[...]) + b_ref[...])


def _affine_relu_packed(x, w, b, bm):
    m, k = x.shape
    n = w.shape[1]
    return pl.pallas_call(
        _affine_relu_body,
        grid=(m // bm,),
        in_specs=[pl.BlockSpec((bm, k), lambda i: (i, 0)),
                  pl.BlockSpec((k, n), lambda i: (0, 0)),
                  pl.BlockSpec((1, n), lambda i: (0, 0))],
        out_specs=pl.BlockSpec((bm, n), lambda i: (i, 0)),
        out_shape=jax.ShapeDtypeStruct((m, n), jnp.float32),
    )(x, w, b)


def _msg_body(u_ref, hs_ref, sk_ref, gk_ref, bdbe2_ref, o_ref):
    u = u_ref[...]
    x = hs_ref[...]
    us = _dot(u, sk_ref[...])
    xg = _dot(x, gk_ref[...])
    acc = _dot(x, bdbe2_ref[...])
    for k in range(H):
        acc = acc + us[:, k * 128:(k + 1) * 128] * xg[:, k * 128:(k + 1) * 128]
    o_ref[...] = acc


def _msg(u_p, hs_p, sk, gk, bdbe2):
    return pl.pallas_call(
        _msg_body,
        grid=(_EP // _BE,),
        in_specs=[pl.BlockSpec((_BE, 128), lambda i: (i, 0)),
                  pl.BlockSpec((_BE, 128), lambda i: (i, 0)),
                  pl.BlockSpec((128, H * 128), lambda i: (0, 0)),
                  pl.BlockSpec((128, H * 128), lambda i: (0, 0)),
                  pl.BlockSpec((128, 128), lambda i: (0, 0))],
        out_specs=pl.BlockSpec((_BE, 128), lambda i: (i, 0)),
        out_shape=jax.ShapeDtypeStruct((_EP, 128), jnp.float32),
    )(u_p, hs_p, sk, gk, bdbe2)


def _gru_body(parts_ref, hid_ref, wi_ref, wh_ref, bi_ref, bh_ref,
              bnn_ref, o_ref):
    m = jax.nn.relu(parts_ref[0] + parts_ref[1] + bnn_ref[...])
    hid = hid_ref[...]
    gi = _dot(m, wi_ref[...]) + bi_ref[...]
    gh = _dot(hid, wh_ref[...]) + bh_ref[...]
    r = jax.nn.sigmoid(gi[:, 0:128] + gh[:, 0:128])
    z = jax.nn.sigmoid(gi[:, 128:256] + gh[:, 128:256])
    ng = jnp.tanh(gi[:, 256:384] + r * gh[:, 256:384])
    o_ref[...] = (1.0 - z) * ng + z * hid


def _gru(parts_p3, hid_p, bdi, bdh, bi3, bh3, bnn128):
    return pl.pallas_call(
        _gru_body,
        grid=(1,),
        in_specs=[pl.BlockSpec((2, _NP, 128), lambda i: (0, 0, 0)),
                  pl.BlockSpec((_NP, 128), lambda i: (0, 0)),
                  pl.BlockSpec((128, 384), lambda i: (0, 0)),
                  pl.BlockSpec((128, 384), lambda i: (0, 0)),
                  pl.BlockSpec((1, 384), lambda i: (0, 0)),
                  pl.BlockSpec((1, 384), lambda i: (0, 0)),
                  pl.BlockSpec((1, 128), lambda i: (0, 0))],
        out_specs=pl.BlockSpec((_NP, 128), lambda i: (0, 0)),
        out_shape=jax.ShapeDtypeStruct((_NP, 128), jnp.float32),
    )(parts_p3, hid_p, bdi, bdh, bi3, bh3, bnn128)


def _readout_body(h_ref, fold_ref, w0_ref, b0_ref, w1_ref, b1_ref, w2_ref,
                  b2_ref, y_ref):
    s = jnp.sum(h_ref[...], axis=0, keepdims=True)
    hg = _dot(s, fold_ref[...]) * (1.0 / N)
    y = jax.nn.relu(_dot(hg, w0_ref[...]) + b0_ref[...])
    y = jax.nn.relu(_dot(y, w1_ref[...]) + b1_ref[...])
    y_ref[...] = _dot(y, w2_ref[...]) + b2_ref[...]


def _readout(h_p, fold, w0, b0, w1, b1, w2, b2):
    return pl.pallas_call(
        _readout_body,
        grid=(1,),
        in_specs=[pl.BlockSpec((_NP, 128), lambda i: (0, 0)),
                  pl.BlockSpec((128, H), lambda i: (0, 0)),
                  pl.BlockSpec(w0.shape, lambda i: (0, 0)),
                  pl.BlockSpec((1, w0.shape[1]), lambda i: (0, 0)),
                  pl.BlockSpec(w1.shape, lambda i: (0, 0)),
                  pl.BlockSpec((1, w1.shape[1]), lambda i: (0, 0)),
                  pl.BlockSpec(w2.shape, lambda i: (0, 0)),
                  pl.BlockSpec((1, w2.shape[1]), lambda i: (0, 0))],
        out_specs=pl.BlockSpec((1, 2), lambda i: (0, 0)),
        out_shape=jax.ShapeDtypeStruct((1, 2), jnp.float32),
    )(h_p, fold, w0, b0.reshape(1, -1), w1, b1.reshape(1, -1), w2,
      b2.reshape(1, -1))


_MESH = plsc.VectorSubcoreMesh(core_axis_name="c", subcore_axis_name="s")
_SC_PARAMS = pltpu.CompilerParams(use_tc_tiling_on_sc=False)


@functools.partial(
    pl.kernel,
    out_type=jax.ShapeDtypeStruct((E, H), jnp.float32),
    mesh=_MESH,
    compiler_params=_SC_PARAMS,
    scratch_types=[pltpu.VMEM((_RPW, _CHUNK), jnp.int32),
                   pltpu.VMEM((_EW, H), jnp.float32),
                   pltpu.VMEM((1, _CHUNK), jnp.int32),
                   pltpu.VMEM((_CHUNK, H), jnp.float32),
                   pltpu.SemaphoreType.DMA,
                   pltpu.SemaphoreType.DMA],
)
def _sc_gather(h_hbm, src_hbm, out_hbm, idx_v, rows_v, idx_x, rows_x,
               sem, sem_x):
    wid = lax.axis_index("s") * 2 + lax.axis_index("c")
    base = wid * _RPW

    pltpu.sync_copy(src_hbm.at[pl.ds(base, _RPW)], idx_v)

    def fire(j, carry):
        pltpu.async_copy(h_hbm.at[idx_v.at[j]],
                         rows_v.at[pl.ds(j * _CHUNK, _CHUNK)], sem)
        return carry

    lax.fori_loop(0, _RPW, fire, 0)

    @pl.when(wid < _XTRA)
    def _():
        r = _RPW * _NW + wid
        pltpu.sync_copy(src_hbm.at[pl.ds(r, 1)], idx_x)
        pltpu.async_copy(h_hbm.at[idx_x.at[0]], rows_x, sem_x)

    pltpu.make_async_copy(out_hbm.at[pl.ds(0, _EW)], rows_v, sem).wait()
    pltpu.sync_copy(rows_v, out_hbm.at[pl.ds(base * _CHUNK, _EW)])

    @pl.when(wid < _XTRA)
    def _():
        r = _RPW * _NW + wid
        pltpu.make_async_copy(out_hbm.at[pl.ds(0, _CHUNK)], rows_x,
                              sem_x).wait()
        pltpu.sync_copy(rows_x, out_hbm.at[pl.ds(r * _CHUNK, _CHUNK)])


@functools.partial(
    pl.kernel,
    out_type=jax.ShapeDtypeStruct((2 * N, H), jnp.float32),
    mesh=_MESH,
    compiler_params=_SC_PARAMS,
    scratch_types=[pltpu.VMEM((_RPW, _CHUNK), jnp.int32),
                   pltpu.VMEM((_EW, H), jnp.float32),
                   pltpu.VMEM((1, _CHUNK), jnp.int32),
                   pltpu.VMEM((_CHUNK, H), jnp.float32),
                   pltpu.VMEM_SHARED((N, H), jnp.float32),
                   pltpu.SemaphoreType.DMA,
                   pltpu.SemaphoreType.DMA],
)
def _sc_scatter(msg_hbm, dst_hbm, zeros_hbm, out_hbm, idx_v, msg_v, idx_x,
                msg_x, acc_sh, sem, sem_x):
    c = lax.axis_index("c")
    s = lax.axis_index("s")
    wid = s * 2 + c
    base = wid * _RPW

    pltpu.async_copy(zeros_hbm.at[pl.ds(s * _NPS, _NPS)],
                     acc_sh.at[pl.ds(s * _NPS, _NPS)], sem_x)
    pltpu.sync_copy(dst_hbm.at[pl.ds(base, _RPW)], idx_v)
    pltpu.sync_copy(msg_hbm.at[pl.ds(base * _CHUNK, _EW)], msg_v)
    pltpu.make_async_copy(zeros_hbm.at[pl.ds(0, _NPS)],
                          acc_sh.at[pl.ds(0, _NPS)], sem_x).wait()
    plsc.subcore_barrier()

    def fire(j, carry):
        pltpu.async_copy(msg_v.at[pl.ds(j * _CHUNK, _CHUNK)],
                         acc_sh.at[idx_v.at[j]], sem, add=True)
        return carry

    lax.fori_loop(0, _RPW, fire, 0)

    @pl.when(wid < _XTRA)
    def _():
        r = _RPW * _NW + wid
        pltpu.sync_copy(dst_hbm.at[pl.ds(r, 1)], idx_x)
        pltpu.sync_copy(msg_hbm.at[pl.ds(r * _CHUNK, _CHUNK)], msg_x)
        pltpu.async_copy(msg_x, acc_sh.at[idx_x.at[0]], sem, add=True)

    pltpu.make_async_copy(msg_hbm.at[pl.ds(0, _EW)], msg_v, sem).wait()

    @pl.when(wid < _XTRA)
    def _():
        pltpu.make_async_copy(msg_hbm.at[pl.ds(0, _CHUNK)], msg_x,
                              sem).wait()

    plsc.subcore_barrier()
    pltpu.sync_copy(acc_sh.at[pl.ds(s * _NPS, _NPS)],
                    out_hbm.at[pl.ds(c * N + s * _NPS, _NPS)])


def kernel(node_feats, edge_feats, edge_index, W_proj, b_proj, We1, be1,
           We2, be2, b_nn, W_ih, W_hh, b_ih, b_hh, W0, b0, W1, b1, W2, b2):
    f32 = jnp.float32
    src2d = edge_index[0].reshape(_ROWS, _CHUNK)
    dst2d = edge_index[1].reshape(_ROWS, _CHUNK)
    zeros = jnp.zeros((N, H), f32)
    eye8 = jnp.asarray(_EYE8)
    sk = jnp.asarray(_SK)
    fold = jnp.asarray(_FOLD)

    we2r = We2.reshape(H, H, H)
    gk = jnp.einsum('ab,kio->aikbo', eye8, we2r).reshape(128, H * 128)
    bdbe2 = jnp.einsum('ab,io->aibo', eye8,
                       be2.reshape(H, H)).reshape(128, 128)

    wi3 = W_ih.reshape(3, H, H)
    wh3 = W_hh.reshape(3, H, H)
    bdi = jnp.einsum('ab,goi->aigbo', eye8, wi3).reshape(128, 384)
    bdh = jnp.einsum('ab,goi->aigbo', eye8, wh3).reshape(128, 384)
    bi3 = _tile8(b_ih, 3)
    bh3 = _tile8(b_hh, 3)
    bnn128 = _tile8(b_nn, 1)

    bdn = jnp.einsum('ab,ko->akbo', eye8, W_proj).reshape(1024, 128)
    bde = jnp.einsum('ab,ko->akbo', eye8, We1).reshape(128, 128)
    bproj128 = _tile8(b_proj, 1)
    be1_128 = _tile8(be1, 1)

    nf8 = node_feats.reshape(_NP, 8 * 128)
    ef8 = edge_feats.reshape(_EP, 128)

    h_p = _affine_relu_packed(nf8, bdn, bproj128, _NP)
    u_p = _affine_relu_packed(ef8, bde, be1_128, 4000)
    hid_p = h_p
    for _ in range(NSTEPS):
        h_rows = h_p.reshape(N, H)
        h_src = _sc_gather(h_rows, src2d)
        msg_p = _msg(u_p, h_src.reshape(_EP, 128), sk, gk, bdbe2)
        parts = _sc_scatter(msg_p.reshape(E, H), dst2d, zeros)
        hid_p = _gru(parts.reshape(2, _NP, 128), hid_p, bdi, bdh, bi3,
                     bh3, bnn128)
        h_p = hid_p
    return _readout(h_p, fold, W0, b0, W1, b1, W2, b2)
